# Initial kernel scaffold; baseline (speedup 1.0000x reference)
import jax
import jax.numpy as jnp
from jax.experimental import pallas as pl


def kernel(inds, mask, table, W_mp, b_mp, w_sc, b_sc):
    raise NotImplementedError("write your pallas kernel here")


# speedup vs baseline: 1.3996x; 1.3996x over previous
import functools

import jax
import jax.numpy as jnp
from jax import lax
from jax.experimental import pallas as pl
from jax.experimental.pallas import tpu as pltpu
from jax.experimental.pallas import tpu_sc as plsc

_NC = 2
_NS = 16
_NW = _NC * _NS
_LANES = 16


def _sc_reductions(inds2, mask0, table, B, L, D):
    BPW = B // _NW
    H = L // 2
    NCH = D // _LANES

    mesh = plsc.VectorSubcoreMesh(core_axis_name="c", subcore_axis_name="s",
                                  num_cores=_NC, num_subcores=_NS)

    @functools.partial(
        pl.kernel,
        out_type=[jax.ShapeDtypeStruct((B, D), jnp.float32)] * 3,
        mesh=mesh,
        compiler_params=pltpu.CompilerParams(use_tc_tiling_on_sc=False,
                                             needs_layout_passes=False),
        scratch_types=[
            pltpu.VMEM((BPW, 2, H), jnp.int32),
            pltpu.VMEM((BPW, L), jnp.float32),
            pltpu.VMEM((2, L, D), jnp.float32),
            pltpu.VMEM((BPW, D), jnp.float32),
            pltpu.VMEM((BPW, D), jnp.float32),
            pltpu.VMEM((BPW, D), jnp.float32),
            pltpu.SemaphoreType.DMA,
            pltpu.SemaphoreType.DMA,
        ],
    )
    def sc_kernel(inds_hbm, mask_hbm, table_hbm, s1_hbm, s2_hbm, e0_hbm,
                  idx_v, mask_v, rows_v, s1_v, s2_v, e0_v, sem0, sem1):
        wid = lax.axis_index("s") * _NC + lax.axis_index("c")
        base = wid * BPW
        pltpu.sync_copy(inds_hbm.at[pl.ds(base, BPW)], idx_v)
        pltpu.sync_copy(mask_hbm.at[pl.ds(base, BPW)], mask_v)

        sems = (sem0, sem1)

        def gather_descs(b, slot):
            return (
                pltpu.make_async_copy(table_hbm.at[idx_v.at[b, 0]],
                                      rows_v.at[slot, pl.ds(0, H)], sems[slot]),
                pltpu.make_async_copy(table_hbm.at[idx_v.at[b, 1]],
                                      rows_v.at[slot, pl.ds(H, H)], sems[slot]),
            )

        def start_g(b, slot):
            for c in gather_descs(b, slot):
                c.start()

        def wait_g(b, slot):
            for c in gather_descs(b, slot):
                c.wait()

        iota = lax.iota(jnp.int32, _LANES)
        zero = jnp.zeros((_LANES,), jnp.float32)

        def compute(b, slot):
            rows = rows_v.at[slot]

            def body(l, carry):
                lv = jnp.full((_LANES,), l, jnp.int32)
                bv = jnp.full((_LANES,), b, jnp.int32)
                m = plsc.load_gather(mask_v, [bv, lv])
                out = []
                for j in range(NCH):
                    r = rows[l, pl.ds(j * _LANES, _LANES)]
                    out.append(carry[j] + r)
                    out.append(carry[NCH + j] + m * r)
                return tuple(out[0::2]) + tuple(out[1::2])

            acc = lax.fori_loop(0, L, body, (zero,) * (2 * NCH))
            for j in range(NCH):
                s1_v[b, pl.ds(j * _LANES, _LANES)] = acc[j]
                s2_v[b, pl.ds(j * _LANES, _LANES)] = acc[NCH + j]
                e0_v[b, pl.ds(j * _LANES, _LANES)] = rows[0, pl.ds(j * _LANES, _LANES)]

        start_g(0, 0)
        start_g(1, 1)

        @pl.loop(0, BPW // 2)
        def _(i):
            for j in range(2):
                b = i * 2 + j
                wait_g(b, j)
                compute(b, j)

                @pl.when(b + 2 < BPW)
                def _():
                    start_g(b + 2, j)

        pltpu.sync_copy(s1_v, s1_hbm.at[pl.ds(base, BPW)])
        pltpu.sync_copy(s2_v, s2_hbm.at[pl.ds(base, BPW)])
        pltpu.sync_copy(e0_v, e0_hbm.at[pl.ds(base, BPW)])

    return sc_kernel(inds2, mask0, table)


def _tc_tail(S1, S2, e0, mask0, W_mp, b_mp2, w_sc2, b_sc2, B, L, D):
    BB = 512
    grid = (B // BB,)

    def body(s1_ref, s2_ref, e0_ref, mask_ref, wmp_ref, bmp_ref, wsc_ref,
             bsc_ref, out_ref):
        s1 = s1_ref[...]
        pooled = s1 * (1.0 / L)
        msg = jnp.tanh(
            jnp.dot(pooled, wmp_ref[...], preferred_element_type=jnp.float32)
            + bmp_ref[...])
        msum = jnp.sum(mask_ref[...], axis=1, keepdims=True)
        id_em = e0_ref[...] + msg
        agg = (s2_ref[...] + msum * msg) / (msum + 1e-8)
        w = wsc_ref[...]
        scores = (
            jnp.dot(id_em, w[:D], preferred_element_type=jnp.float32)
            + jnp.dot(agg, w[D:], preferred_element_type=jnp.float32)
            + bsc_ref[0, 0])
        out_ref[...] = scores

    blk = lambda r, c: pl.BlockSpec((r, c), lambda i: (i, 0))
    full = lambda r, c: pl.BlockSpec((r, c), lambda i: (0, 0))
    out = pl.pallas_call(
        body,
        grid=grid,
        in_specs=[blk(BB, D), blk(BB, D), blk(BB, D), blk(BB, L),
                  full(D, D), full(1, D), full(2 * D, 1), full(1, 1)],
        out_specs=pl.BlockSpec((BB, 1), lambda i: (i, 0)),
        out_shape=jax.ShapeDtypeStruct((B, 1), jnp.float32),
    )(S1, S2, e0, mask0, W_mp, b_mp2, w_sc2, b_sc2)
    return out[:, 0]


def kernel(inds, mask, table, W_mp, b_mp, w_sc, b_sc):
    B, L = inds.shape
    V, D = table.shape
    inds2 = inds.astype(jnp.int32).reshape(B, 2, L // 2)
    mask0 = mask.at[:, 0].set(0.0)
    S1, S2, e0 = _sc_reductions(inds2, mask0, table, B, L, D)
    b_mp2 = b_mp.reshape(1, D)
    w_sc2 = w_sc.reshape(2 * D, 1)
    b_sc2 = b_sc.reshape(1, 1)
    return _tc_tail(S1, S2, e0, mask0, W_mp, b_mp2, w_sc2, b_sc2, B, L, D)
